# baseline (speedup 1.0000x reference)
import functools

import jax
import jax.numpy as jnp
from jax import lax
from jax.experimental import pallas as pl
from jax.experimental.pallas import tpu as pltpu
from jax.experimental.pallas import tpu_sc as plsc

NUM_ENTITY = 1000000
NUM_RELATION = 1000
EMB_DIM = 64
BATCH = 16384

NC = 2
NS = 16
NW = NC * NS
BPW = BATCH // NW
CHUNK = 128
NCHUNK = BPW // CHUNK

_mesh = plsc.VectorSubcoreMesh(core_axis_name="c", subcore_axis_name="s")


@functools.partial(
    pl.kernel,
    mesh=_mesh,
    compiler_params=pltpu.CompilerParams(use_tc_tiling_on_sc=False),
    out_type=[
        jax.ShapeDtypeStruct((BATCH, EMB_DIM), jnp.float32),
        jax.ShapeDtypeStruct((BATCH, EMB_DIM), jnp.float32),
        jax.ShapeDtypeStruct((BATCH, EMB_DIM), jnp.float32),
    ],
    scratch_types=[
        pltpu.VMEM((BPW,), jnp.int32),
        pltpu.VMEM((BPW,), jnp.int32),
        pltpu.VMEM((BPW,), jnp.int32),
        pltpu.VMEM((BPW, EMB_DIM), jnp.float32),
        pltpu.VMEM((BPW, EMB_DIM), jnp.float32),
        pltpu.VMEM((BPW, EMB_DIM), jnp.float32),
        pltpu.SemaphoreType.DMA,
        pltpu.SemaphoreType.DMA,
        pltpu.SemaphoreType.DMA,
    ],
)
def _transe_lookup(head_hbm, rel_hbm, tail_hbm, ent_hbm, rel_tab_hbm,
                   head_out, rel_out, tail_out,
                   hidx, ridx, tidx, hrows, rrows, trows,
                   sem_h, sem_r, sem_t):
    wid = lax.axis_index("s") * NC + lax.axis_index("c")
    base = wid * BPW
    sl = pl.ds(base, BPW)

    pltpu.sync_copy(head_hbm.at[sl], hidx)
    pltpu.sync_copy(rel_hbm.at[sl], ridx)
    pltpu.sync_copy(tail_hbm.at[sl], tidx)

    hcopies = []
    rcopies = []
    tcopies = []
    for j in range(NCHUNK):
        cs = pl.ds(j * CHUNK, CHUNK)
        hcopies.append(
            pltpu.async_copy(ent_hbm.at[hidx.at[cs]], hrows.at[cs], sem_h))
        rcopies.append(
            pltpu.async_copy(rel_tab_hbm.at[ridx.at[cs]], rrows.at[cs], sem_r))
        tcopies.append(
            pltpu.async_copy(ent_hbm.at[tidx.at[cs]], trows.at[cs], sem_t))

    for c in hcopies:
        c.wait()
    pltpu.sync_copy(hrows, head_out.at[sl])
    for c in rcopies:
        c.wait()
    pltpu.sync_copy(rrows, rel_out.at[sl])
    for c in tcopies:
        c.wait()
    pltpu.sync_copy(trows, tail_out.at[sl])


def kernel(head, relation, tail, entity_table, relation_table):
    head = head.astype(jnp.int32)
    relation = relation.astype(jnp.int32)
    tail = tail.astype(jnp.int32)
    return tuple(_transe_lookup(head, relation, tail,
                                entity_table, relation_table))
